# baseline (device time: 697850 ns/iter reference)
import jax
import jax.numpy as jnp
from jax import lax
from jax.experimental import pallas as pl
from jax.experimental.pallas import tpu as pltpu

N_DEV = 16
B, SQ, D = 4, 256, 1024
H_PER = 8
DH = 128
ROWS = B * SQ
SCALE = 0.08838834764831843

QR = ROWS // 4
HALF = D // 2
ZC = 2 * QR // 4
BH = HALF // 2


def _allreduce(partial):

    def body(p_ref, out_ref, stage_a, abuf_p, abuf_m, zbuf, bstage,
             bbuf_p, bbuf_m, cbuf_p, cbuf_m, send_sems, recv_sems):
        my = lax.axis_index("i")
        z = my // 4
        q = my % 4
        p_right = z * 4 + (q + 1) % 4
        p_left = z * 4 + (q - 1) % 4
        z_right = ((z + 1) % 4) * 4 + q
        z_left = ((z - 1) % 4) * 4 + q

        barrier_sem = pltpu.get_barrier_semaphore()
        for nbr in (p_right, p_left, z_right, z_left):
            pl.semaphore_signal(
                barrier_sem, inc=1,
                device_id=(nbr,), device_id_type=pl.DeviceIdType.MESH,
            )
        pl.semaphore_wait(barrier_sem, 4)

        def rdma_start(src, dst, sem_idx, dev):
            r = pltpu.make_async_remote_copy(
                src_ref=src, dst_ref=dst,
                send_sem=send_sems.at[sem_idx],
                recv_sem=recv_sems.at[sem_idx],
                device_id=(dev,), device_id_type=pl.DeviceIdType.MESH,
            )
            r.start()
            return r

        stage_a[0, :, :] = p_ref[pl.ds(q * QR, QR), 0:HALF]
        stage_a[1, :, :] = p_ref[pl.ds(((q + 1) % 4) * QR, QR), HALF:D]
        for s in range(3):
            src_p = stage_a.at[0] if s == 0 else abuf_p.at[s - 1]
            src_m = stage_a.at[1] if s == 0 else abuf_m.at[s - 1]
            r1 = rdma_start(src_p, abuf_p.at[s], 0 + s, p_right)
            r2 = rdma_start(src_m, abuf_m.at[s], 3 + s, p_left)
            r1.wait()
            r2.wait()
            rc_p = (q - 1 - s) % 4
            rc_m = (q + 2 + s) % 4
            abuf_p[s, :, :] = abuf_p[s, :, :] + p_ref[pl.ds(rc_p * QR, QR), 0:HALF]
            abuf_m[s, :, :] = abuf_m[s, :, :] + p_ref[pl.ds(rc_m * QR, QR), HALF:D]

        zbuf[0:QR, :] = abuf_p[2, :, :]
        zbuf[QR:2 * QR, :] = abuf_m[2, :, :]

        bstage[0, :, :] = zbuf[pl.ds(z * ZC, ZC), 0:BH]
        bstage[1, :, :] = zbuf[pl.ds(((z + 1) % 4) * ZC, ZC), BH:HALF]
        for s in range(3):
            src_p = bstage.at[0] if s == 0 else bbuf_p.at[s - 1]
            src_m = bstage.at[1] if s == 0 else bbuf_m.at[s - 1]
            r1 = rdma_start(src_p, bbuf_p.at[s], 6 + s, z_right)
            r2 = rdma_start(src_m, bbuf_m.at[s], 9 + s, z_left)
            r1.wait()
            r2.wait()
            rc_p = (z - 1 - s) % 4
            rc_m = (z + 2 + s) % 4
            bbuf_p[s, :, :] = bbuf_p[s, :, :] + zbuf[pl.ds(rc_p * ZC, ZC), 0:BH]
            bbuf_m[s, :, :] = bbuf_m[s, :, :] + zbuf[pl.ds(rc_m * ZC, ZC), BH:HALF]
        zbuf[pl.ds(((z + 1) % 4) * ZC, ZC), 0:BH] = bbuf_p[2, :, :]
        zbuf[pl.ds(z * ZC, ZC), BH:HALF] = bbuf_m[2, :, :]
        for s in range(3):
            src_p = bbuf_p.at[2] if s == 0 else bbuf_p.at[2 + s]
            src_m = bbuf_m.at[2] if s == 0 else bbuf_m.at[2 + s]
            r1 = rdma_start(src_p, bbuf_p.at[3 + s], 12 + s, z_right)
            r2 = rdma_start(src_m, bbuf_m.at[3 + s], 15 + s, z_left)
            r1.wait()
            r2.wait()
            zbuf[pl.ds(((z - s) % 4) * ZC, ZC), 0:BH] = bbuf_p[3 + s, :, :]
            zbuf[pl.ds(((z + 1 + s) % 4) * ZC, ZC), BH:HALF] = bbuf_m[3 + s, :, :]

        out_ref[pl.ds(((q + 1) % 4) * QR, QR), 0:HALF] = zbuf[0:QR, :]
        out_ref[pl.ds(q * QR, QR), HALF:D] = zbuf[QR:2 * QR, :]

        for s in range(3):
            src_p = zbuf.at[pl.ds(0, QR)] if s == 0 else cbuf_p.at[s - 1]
            src_m = zbuf.at[pl.ds(QR, QR)] if s == 0 else cbuf_m.at[s - 1]
            r1 = rdma_start(src_p, cbuf_p.at[s], 18 + s, p_right)
            r2 = rdma_start(src_m, cbuf_m.at[s], 21 + s, p_left)
            r1.wait()
            r2.wait()
            out_ref[pl.ds(((q - s) % 4) * QR, QR), 0:HALF] = cbuf_p[s, :, :]
            out_ref[pl.ds(((q + 1 + s) % 4) * QR, QR), HALF:D] = cbuf_m[s, :, :]

    return pl.pallas_call(
        body,
        out_shape=jax.ShapeDtypeStruct((ROWS, D), jnp.float32),
        in_specs=[pl.BlockSpec(memory_space=pltpu.VMEM)],
        out_specs=pl.BlockSpec(memory_space=pltpu.VMEM),
        scratch_shapes=[
            pltpu.VMEM((2, QR, HALF), jnp.float32),
            pltpu.VMEM((3, QR, HALF), jnp.float32),
            pltpu.VMEM((3, QR, HALF), jnp.float32),
            pltpu.VMEM((2 * QR, HALF), jnp.float32),
            pltpu.VMEM((2, ZC, BH), jnp.float32),
            pltpu.VMEM((6, ZC, BH), jnp.float32),
            pltpu.VMEM((6, ZC, BH), jnp.float32),
            pltpu.VMEM((3, QR, HALF), jnp.float32),
            pltpu.VMEM((3, QR, HALF), jnp.float32),
            pltpu.SemaphoreType.DMA((24,)),
            pltpu.SemaphoreType.DMA((24,)),
        ],
        compiler_params=pltpu.CompilerParams(collective_id=0),
    )(partial)


SKV = 1024


def _attention(my, Q4, K_ext, V_ext):

    def body(my_ref, q_ref, k_ref, v_ref, o_ref):
        del my_ref
        qq = q_ref[:, :]
        kk = k_ref[0, :, :]
        vv = v_ref[0, :, :]
        s = lax.dot_general(
            qq, kk, (((1,), (1,)), ((), ())),
            preferred_element_type=jnp.float32,
        ) * SCALE
        m = jnp.max(s, axis=1, keepdims=True)
        e = jnp.exp(s - m)
        l = jnp.sum(e, axis=1, keepdims=True)
        o = jnp.dot(e, vv, preferred_element_type=jnp.float32)
        o_ref[:, :] = o / l

    grid_spec = pltpu.PrefetchScalarGridSpec(
        num_scalar_prefetch=1,
        grid=(B, H_PER),
        in_specs=[
            pl.BlockSpec((SQ, DH), lambda b, h, my_ref: (b, h)),
            pl.BlockSpec(
                (1, SKV, DH),
                lambda b, h, my_ref: (b, 0, my_ref[0] * H_PER + h),
            ),
            pl.BlockSpec(
                (1, SKV, DH),
                lambda b, h, my_ref: (b, 0, my_ref[0] * H_PER + h),
            ),
        ],
        out_specs=pl.BlockSpec((SQ, DH), lambda b, h, my_ref: (b, h)),
    )
    return pl.pallas_call(
        body,
        grid_spec=grid_spec,
        out_shape=jax.ShapeDtypeStruct((B * SQ, H_PER * DH), jnp.float32),
    )(
        my,
        Q4,
        K_ext.reshape(B, SKV, -1),
        V_ext.reshape(B, SKV, -1),
    )


def kernel(x, Wq, Wo, K_ext, V_ext):
    my = lax.axis_index("i")

    Q = x.reshape(B * SQ, D) @ Wq

    attn = _attention(jnp.reshape(my, (1,)).astype(jnp.int32), Q, K_ext, V_ext)

    partial = attn @ Wo

    return _allreduce(partial).reshape(B, SQ, D)


# device time: 122236 ns/iter; 5.7090x vs baseline; 5.7090x over previous
import jax
import jax.numpy as jnp
from jax import lax
from jax.experimental import pallas as pl
from jax.experimental.pallas import tpu as pltpu

N_DEV = 16
B, SQ, D = 4, 256, 1024
H_PER = 8
DH = 128
ROWS = B * SQ
SCALE = 0.08838834764831843

QR = ROWS // 4
HALF = D // 2
ZC = 2 * QR // 4
BH = HALF // 2


def _allreduce(partial):

    def body(p_ref, out_ref, stage_a, abuf_p, abuf_m, zbuf, bstage,
             bbuf_p, bbuf_m, cbuf_p, cbuf_m, send_sems, recv_sems):
        my = lax.axis_index("i")
        z = my // 4
        q = my % 4
        p_right = z * 4 + (q + 1) % 4
        p_left = z * 4 + (q - 1) % 4
        z_right = ((z + 1) % 4) * 4 + q
        z_left = ((z - 1) % 4) * 4 + q

        barrier_sem = pltpu.get_barrier_semaphore()
        for nbr in (p_right, p_left, z_right, z_left):
            pl.semaphore_signal(
                barrier_sem, inc=1,
                device_id=(nbr,), device_id_type=pl.DeviceIdType.MESH,
            )
        pl.semaphore_wait(barrier_sem, 4)

        def rdma_start(src, dst, sem_idx, dev):
            r = pltpu.make_async_remote_copy(
                src_ref=src, dst_ref=dst,
                send_sem=send_sems.at[sem_idx],
                recv_sem=recv_sems.at[sem_idx],
                device_id=(dev,), device_id_type=pl.DeviceIdType.MESH,
            )
            r.start()
            return r

        stage_a[0, :, :] = p_ref[pl.ds(q * QR, QR), 0:HALF]
        stage_a[1, :, :] = p_ref[pl.ds(((q + 1) % 4) * QR, QR), HALF:D]
        for s in range(3):
            src_p = stage_a.at[0] if s == 0 else abuf_p.at[s - 1]
            src_m = stage_a.at[1] if s == 0 else abuf_m.at[s - 1]
            r1 = rdma_start(src_p, abuf_p.at[s], 0 + s, p_right)
            r2 = rdma_start(src_m, abuf_m.at[s], 3 + s, p_left)
            r1.wait()
            r2.wait()
            rc_p = (q - 1 - s) % 4
            rc_m = (q + 2 + s) % 4
            abuf_p[s, :, :] = abuf_p[s, :, :] + p_ref[pl.ds(rc_p * QR, QR), 0:HALF]
            abuf_m[s, :, :] = abuf_m[s, :, :] + p_ref[pl.ds(rc_m * QR, QR), HALF:D]

        zbuf[0:QR, :] = abuf_p[2, :, :]
        zbuf[QR:2 * QR, :] = abuf_m[2, :, :]

        bstage[0, :, :] = zbuf[pl.ds(z * ZC, ZC), 0:BH]
        bstage[1, :, :] = zbuf[pl.ds(((z + 1) % 4) * ZC, ZC), BH:HALF]
        for s in range(3):
            src_p = bstage.at[0] if s == 0 else bbuf_p.at[s - 1]
            src_m = bstage.at[1] if s == 0 else bbuf_m.at[s - 1]
            r1 = rdma_start(src_p, bbuf_p.at[s], 6 + s, z_right)
            r2 = rdma_start(src_m, bbuf_m.at[s], 9 + s, z_left)
            r1.wait()
            r2.wait()
            rc_p = (z - 1 - s) % 4
            rc_m = (z + 2 + s) % 4
            bbuf_p[s, :, :] = bbuf_p[s, :, :] + zbuf[pl.ds(rc_p * ZC, ZC), 0:BH]
            bbuf_m[s, :, :] = bbuf_m[s, :, :] + zbuf[pl.ds(rc_m * ZC, ZC), BH:HALF]
        zbuf[pl.ds(((z + 1) % 4) * ZC, ZC), 0:BH] = bbuf_p[2, :, :]
        zbuf[pl.ds(z * ZC, ZC), BH:HALF] = bbuf_m[2, :, :]
        for s in range(3):
            src_p = bbuf_p.at[2] if s == 0 else bbuf_p.at[2 + s]
            src_m = bbuf_m.at[2] if s == 0 else bbuf_m.at[2 + s]
            r1 = rdma_start(src_p, bbuf_p.at[3 + s], 12 + s, z_right)
            r2 = rdma_start(src_m, bbuf_m.at[3 + s], 15 + s, z_left)
            r1.wait()
            r2.wait()
            zbuf[pl.ds(((z - s) % 4) * ZC, ZC), 0:BH] = bbuf_p[3 + s, :, :]
            zbuf[pl.ds(((z + 1 + s) % 4) * ZC, ZC), BH:HALF] = bbuf_m[3 + s, :, :]

        out_ref[pl.ds(((q + 1) % 4) * QR, QR), 0:HALF] = zbuf[0:QR, :]
        out_ref[pl.ds(q * QR, QR), HALF:D] = zbuf[QR:2 * QR, :]

        for s in range(3):
            src_p = zbuf.at[pl.ds(0, QR)] if s == 0 else cbuf_p.at[s - 1]
            src_m = zbuf.at[pl.ds(QR, QR)] if s == 0 else cbuf_m.at[s - 1]
            r1 = rdma_start(src_p, cbuf_p.at[s], 18 + s, p_right)
            r2 = rdma_start(src_m, cbuf_m.at[s], 21 + s, p_left)
            r1.wait()
            r2.wait()
            out_ref[pl.ds(((q - s) % 4) * QR, QR), 0:HALF] = cbuf_p[s, :, :]
            out_ref[pl.ds(((q + 1 + s) % 4) * QR, QR), HALF:D] = cbuf_m[s, :, :]

    return pl.pallas_call(
        body,
        out_shape=jax.ShapeDtypeStruct((ROWS, D), jnp.float32),
        in_specs=[pl.BlockSpec(memory_space=pltpu.VMEM)],
        out_specs=pl.BlockSpec(memory_space=pltpu.VMEM),
        scratch_shapes=[
            pltpu.VMEM((2, QR, HALF), jnp.float32),
            pltpu.VMEM((3, QR, HALF), jnp.float32),
            pltpu.VMEM((3, QR, HALF), jnp.float32),
            pltpu.VMEM((2 * QR, HALF), jnp.float32),
            pltpu.VMEM((2, ZC, BH), jnp.float32),
            pltpu.VMEM((6, ZC, BH), jnp.float32),
            pltpu.VMEM((6, ZC, BH), jnp.float32),
            pltpu.VMEM((3, QR, HALF), jnp.float32),
            pltpu.VMEM((3, QR, HALF), jnp.float32),
            pltpu.SemaphoreType.DMA((24,)),
            pltpu.SemaphoreType.DMA((24,)),
        ],
        compiler_params=pltpu.CompilerParams(collective_id=0),
    )(partial)


SKV = 1024


def _attention(my, Q4, K_ext, V_ext):

    def body(my_ref, q_ref, k_ref, v_ref, o_ref):
        del my_ref
        for h in range(H_PER):
            qq = q_ref[:, h * DH:(h + 1) * DH]
            kk = k_ref[0, :, h, :]
            vv = v_ref[0, :, h, :]
            s = lax.dot_general(
                qq, kk, (((1,), (1,)), ((), ())),
                preferred_element_type=jnp.float32,
            ) * SCALE
            m = jnp.max(s, axis=1, keepdims=True)
            e = jnp.exp(s - m)
            l = jnp.sum(e, axis=1, keepdims=True)
            o = jnp.dot(e, vv, preferred_element_type=jnp.float32)
            o_ref[:, h * DH:(h + 1) * DH] = o / l

    grid_spec = pltpu.PrefetchScalarGridSpec(
        num_scalar_prefetch=1,
        grid=(B,),
        in_specs=[
            pl.BlockSpec((SQ, H_PER * DH), lambda b, my_ref: (b, 0)),
            pl.BlockSpec(
                (1, SKV, H_PER, DH), lambda b, my_ref: (b, 0, my_ref[0], 0)
            ),
            pl.BlockSpec(
                (1, SKV, H_PER, DH), lambda b, my_ref: (b, 0, my_ref[0], 0)
            ),
        ],
        out_specs=pl.BlockSpec((SQ, H_PER * DH), lambda b, my_ref: (b, 0)),
    )
    return pl.pallas_call(
        body,
        grid_spec=grid_spec,
        out_shape=jax.ShapeDtypeStruct((B * SQ, H_PER * DH), jnp.float32),
    )(my, Q4, K_ext, V_ext)


def kernel(x, Wq, Wo, K_ext, V_ext):
    my = lax.axis_index("i")

    Q = x.reshape(B * SQ, D) @ Wq

    attn = _attention(jnp.reshape(my, (1,)).astype(jnp.int32), Q, K_ext, V_ext)

    partial = attn @ Wo

    return _allreduce(partial).reshape(B, SQ, D)


# device time: 116331 ns/iter; 5.9988x vs baseline; 1.0508x over previous
import jax
import jax.numpy as jnp
from jax import lax
from jax.experimental import pallas as pl
from jax.experimental.pallas import tpu as pltpu

N_DEV = 16
B, SQ, D = 4, 256, 1024
H_PER = 8
DH = 128
ROWS = B * SQ
SCALE = 0.08838834764831843

QR = ROWS // 4
HALF = D // 2
ZC = 2 * QR // 4
BH = HALF // 2


CW = D // 4
BW = CW // 2


def _allreduce(partial):

    def body(p_ref, out_ref, stage_a, abuf_p, abuf_m, zbuf, bstage,
             bbuf_p, bbuf_m, cbuf_p, cbuf_m, send_sems, recv_sems):
        my = lax.axis_index("i")
        z = my // 4
        q = my % 4
        p_right = z * 4 + (q + 1) % 4
        p_left = z * 4 + (q - 1) % 4
        z_right = ((z + 1) % 4) * 4 + q
        z_left = ((z - 1) % 4) * 4 + q

        barrier_sem = pltpu.get_barrier_semaphore()
        for nbr in (p_right, p_left, z_right, z_left):
            pl.semaphore_signal(
                barrier_sem, inc=1,
                device_id=(nbr,), device_id_type=pl.DeviceIdType.MESH,
            )
        pl.semaphore_wait(barrier_sem, 4)

        def rdma_start(src, dst, sem_idx, dev):
            r = pltpu.make_async_remote_copy(
                src_ref=src, dst_ref=dst,
                send_sem=send_sems.at[sem_idx],
                recv_sem=recv_sems.at[sem_idx],
                device_id=(dev,), device_id_type=pl.DeviceIdType.MESH,
            )
            r.start()
            return r

        pipes = (
            (0, q, z, p_right, p_left, z_right, z_left, 0, 0),
            (1, z, q, z_right, z_left, p_right, p_left, 2 * CW, 24),
        )

        def glue(pp, s):
            p, pos1, pos2, r1r, r1l, r2r, r2l, c0, sb = pp
            if s == 0:
                stage_a[p, 0, :, :] = p_ref[pl.ds(pos1 * QR, QR), c0:c0 + CW]
                stage_a[p, 1, :, :] = p_ref[
                    pl.ds(((pos1 + 1) % 4) * QR, QR), c0 + CW:c0 + 2 * CW]
            elif s == 3:
                zbuf[p, 0:QR, :] = abuf_p[p, 2, :, :]
                zbuf[p, QR:2 * QR, :] = abuf_m[p, 2, :, :]
                bstage[p, 0, :, :] = zbuf[p, pl.ds(pos2 * ZC, ZC), 0:BW]
                bstage[p, 1, :, :] = zbuf[
                    p, pl.ds(((pos2 + 1) % 4) * ZC, ZC), BW:CW]
            elif s == 6:
                zbuf[p, pl.ds(((pos2 + 1) % 4) * ZC, ZC), 0:BW] = bbuf_p[p, 2, :, :]
                zbuf[p, pl.ds(pos2 * ZC, ZC), BW:CW] = bbuf_m[p, 2, :, :]
            elif s == 9:
                out_ref[pl.ds(((pos1 + 1) % 4) * QR, QR), c0:c0 + CW] = \
                    zbuf[p, 0:QR, :]
                out_ref[pl.ds(pos1 * QR, QR), c0 + CW:c0 + 2 * CW] = \
                    zbuf[p, QR:2 * QR, :]

        def start(pp, s):
            p, pos1, pos2, r1r, r1l, r2r, r2l, c0, sb = pp
            if s < 3:
                src_p = stage_a.at[p, 0] if s == 0 else abuf_p.at[p, s - 1]
                src_m = stage_a.at[p, 1] if s == 0 else abuf_m.at[p, s - 1]
                return [
                    rdma_start(src_p, abuf_p.at[p, s], sb + s, r1r),
                    rdma_start(src_m, abuf_m.at[p, s], sb + 3 + s, r1l),
                ]
            if s < 6:
                t = s - 3
                src_p = bstage.at[p, 0] if t == 0 else bbuf_p.at[p, t - 1]
                src_m = bstage.at[p, 1] if t == 0 else bbuf_m.at[p, t - 1]
                return [
                    rdma_start(src_p, bbuf_p.at[p, t], sb + 6 + t, r2r),
                    rdma_start(src_m, bbuf_m.at[p, t], sb + 9 + t, r2l),
                ]
            if s < 9:
                t = s - 6
                return [
                    rdma_start(bbuf_p.at[p, 2 + t], bbuf_p.at[p, 3 + t],
                               sb + 12 + t, r2r),
                    rdma_start(bbuf_m.at[p, 2 + t], bbuf_m.at[p, 3 + t],
                               sb + 15 + t, r2l),
                ]
            t = s - 9
            src_p = zbuf.at[p, pl.ds(0, QR)] if t == 0 else cbuf_p.at[p, t - 1]
            src_m = zbuf.at[p, pl.ds(QR, QR)] if t == 0 else cbuf_m.at[p, t - 1]
            return [
                rdma_start(src_p, cbuf_p.at[p, t], sb + 18 + t, r1r),
                rdma_start(src_m, cbuf_m.at[p, t], sb + 21 + t, r1l),
            ]

        def finish(pp, s):
            p, pos1, pos2, r1r, r1l, r2r, r2l, c0, sb = pp
            if s < 3:
                rc_p = (pos1 - 1 - s) % 4
                rc_m = (pos1 + 2 + s) % 4
                abuf_p[p, s, :, :] = abuf_p[p, s, :, :] + \
                    p_ref[pl.ds(rc_p * QR, QR), c0:c0 + CW]
                abuf_m[p, s, :, :] = abuf_m[p, s, :, :] + \
                    p_ref[pl.ds(rc_m * QR, QR), c0 + CW:c0 + 2 * CW]
            elif s < 6:
                t = s - 3
                rc_p = (pos2 - 1 - t) % 4
                rc_m = (pos2 + 2 + t) % 4
                bbuf_p[p, t, :, :] = bbuf_p[p, t, :, :] + \
                    zbuf[p, pl.ds(rc_p * ZC, ZC), 0:BW]
                bbuf_m[p, t, :, :] = bbuf_m[p, t, :, :] + \
                    zbuf[p, pl.ds(rc_m * ZC, ZC), BW:CW]
            elif s < 9:
                t = s - 6
                zbuf[p, pl.ds(((pos2 - t) % 4) * ZC, ZC), 0:BW] = \
                    bbuf_p[p, 3 + t, :, :]
                zbuf[p, pl.ds(((pos2 + 1 + t) % 4) * ZC, ZC), BW:CW] = \
                    bbuf_m[p, 3 + t, :, :]
            else:
                t = s - 9
                out_ref[pl.ds(((pos1 - t) % 4) * QR, QR), c0:c0 + CW] = \
                    cbuf_p[p, t, :, :]
                out_ref[pl.ds(((pos1 + 1 + t) % 4) * QR, QR),
                        c0 + CW:c0 + 2 * CW] = cbuf_m[p, t, :, :]

        for s in range(12):
            for pp in pipes:
                glue(pp, s)
            rdmas = start(pipes[0], s) + start(pipes[1], s)
            for r in rdmas:
                r.wait()
            for pp in pipes:
                finish(pp, s)

    return pl.pallas_call(
        body,
        out_shape=jax.ShapeDtypeStruct((ROWS, D), jnp.float32),
        in_specs=[pl.BlockSpec(memory_space=pltpu.VMEM)],
        out_specs=pl.BlockSpec(memory_space=pltpu.VMEM),
        scratch_shapes=[
            pltpu.VMEM((2, 2, QR, CW), jnp.float32),
            pltpu.VMEM((2, 3, QR, CW), jnp.float32),
            pltpu.VMEM((2, 3, QR, CW), jnp.float32),
            pltpu.VMEM((2, 2 * QR, CW), jnp.float32),
            pltpu.VMEM((2, 2, ZC, BW), jnp.float32),
            pltpu.VMEM((2, 6, ZC, BW), jnp.float32),
            pltpu.VMEM((2, 6, ZC, BW), jnp.float32),
            pltpu.VMEM((2, 3, QR, CW), jnp.float32),
            pltpu.VMEM((2, 3, QR, CW), jnp.float32),
            pltpu.SemaphoreType.DMA((48,)),
            pltpu.SemaphoreType.DMA((48,)),
        ],
        compiler_params=pltpu.CompilerParams(collective_id=0),
    )(partial)


SKV = 1024


def _attention(my, Q4, K_ext, V_ext):

    def body(my_ref, q_ref, k_ref, v_ref, o_ref):
        del my_ref
        for h in range(H_PER):
            qq = q_ref[:, h * DH:(h + 1) * DH]
            kk = k_ref[0, :, h, :]
            vv = v_ref[0, :, h, :]
            s = lax.dot_general(
                qq, kk, (((1,), (1,)), ((), ())),
                preferred_element_type=jnp.float32,
            ) * SCALE
            m = jnp.max(s, axis=1, keepdims=True)
            e = jnp.exp(s - m)
            l = jnp.sum(e, axis=1, keepdims=True)
            o = jnp.dot(e, vv, preferred_element_type=jnp.float32)
            o_ref[:, h * DH:(h + 1) * DH] = o / l

    grid_spec = pltpu.PrefetchScalarGridSpec(
        num_scalar_prefetch=1,
        grid=(B,),
        in_specs=[
            pl.BlockSpec((SQ, H_PER * DH), lambda b, my_ref: (b, 0)),
            pl.BlockSpec(
                (1, SKV, H_PER, DH), lambda b, my_ref: (b, 0, my_ref[0], 0)
            ),
            pl.BlockSpec(
                (1, SKV, H_PER, DH), lambda b, my_ref: (b, 0, my_ref[0], 0)
            ),
        ],
        out_specs=pl.BlockSpec((SQ, H_PER * DH), lambda b, my_ref: (b, 0)),
    )
    return pl.pallas_call(
        body,
        grid_spec=grid_spec,
        out_shape=jax.ShapeDtypeStruct((B * SQ, H_PER * DH), jnp.float32),
    )(my, Q4, K_ext, V_ext)


def kernel(x, Wq, Wo, K_ext, V_ext):
    my = lax.axis_index("i")

    Q = x.reshape(B * SQ, D) @ Wq

    attn = _attention(jnp.reshape(my, (1,)).astype(jnp.int32), Q, K_ext, V_ext)

    partial = attn @ Wo

    return _allreduce(partial).reshape(B, SQ, D)
